# baseline (speedup 1.0000x reference)
import functools

import jax
import jax.numpy as jnp
from jax import lax
from jax.experimental import pallas as pl
from jax.experimental.pallas import tpu as pltpu
from jax.experimental.pallas import tpu_sc as plsc

N_NODES = 10000
D = 128
NPAD = 10240
NC = 2
NS = 16
NW = NC * NS
CHUNK = 64
CPW = 160
NBUF = 4
EPW = CPW * CHUNK
EPAD = NW * EPW
ROWS_PER_TILE = NPAD // NS

_MESH = plsc.VectorSubcoreMesh(core_axis_name="c", subcore_axis_name="s")


@functools.partial(
    pl.kernel,
    out_type=jax.ShapeDtypeStruct((NW, NPAD), jnp.float32),
    mesh=_MESH,
    scratch_types=[
        pltpu.VMEM((CPW, CHUNK), jnp.int32),
        pltpu.VMEM((NPAD,), jnp.float32),
    ],
    compiler_params=pltpu.CompilerParams(needs_layout_passes=False),
)
def _deg_kernel(dst_hbm, deg_out, dst_v, hist_v):
    c = lax.axis_index("c")
    s = lax.axis_index("s")
    w = s * NC + c
    pltpu.sync_copy(dst_hbm.at[w], dst_v)

    def _zero(i, carry):
        hist_v[pl.ds(i * 16, 16)] = jnp.zeros((16,), jnp.float32)
        return carry

    lax.fori_loop(0, NPAD // 16, _zero, 0)

    ones = jnp.ones((16,), jnp.float32)

    def _count(i, carry):
        j = i // (CHUNK // 16)
        k = i % (CHUNK // 16)
        idx = dst_v[j, pl.ds(k * 16, 16)]
        plsc.addupdate_scatter(hist_v, [idx], ones)
        return carry

    lax.fori_loop(0, EPW // 16, _count, 0)

    pltpu.sync_copy(hist_v, deg_out.at[w])


NSTAGE = 4
HSTAGE = CPW // NSTAGE


@functools.partial(
    pl.kernel,
    out_type=jax.ShapeDtypeStruct((NC, NPAD, D), jnp.float32),
    mesh=_MESH,
    scratch_types=[
        pltpu.VMEM((HSTAGE, CHUNK), jnp.int32),
        pltpu.VMEM((HSTAGE, CHUNK), jnp.int32),
        pltpu.VMEM((NBUF, CHUNK, D), jnp.float32),
        pltpu.VMEM_SHARED((NPAD, D), jnp.float32),
        pltpu.SemaphoreType.DMA,
        pltpu.SemaphoreType.DMA,
    ],
)
def _agg_kernel(hp_hbm, src_hbm, dst_hbm, zeros_hbm, out_hbm,
                src_v, dst_v, bufs, acc, gsem, ssem):
    c = lax.axis_index("c")
    s = lax.axis_index("s")
    w = s * NC + c

    row0 = s * ROWS_PER_TILE
    pltpu.sync_copy(
        zeros_hbm.at[pl.ds(row0, ROWS_PER_TILE)],
        acc.at[pl.ds(row0, ROWS_PER_TILE)],
    )
    plsc.subcore_barrier()

    def _gstart(j, b):
        pltpu.async_copy(hp_hbm.at[src_v.at[j]], bufs.at[b], gsem)

    def _gwait(j, b):
        pltpu.make_async_copy(hp_hbm.at[src_v.at[j]], bufs.at[b], gsem).wait()

    def _sstart(j, b):
        pltpu.async_copy(bufs.at[b], acc.at[dst_v.at[j]], ssem, add=True)

    def _swait(j, b):
        pltpu.make_async_copy(bufs.at[b], acc.at[dst_v.at[j]], ssem).wait()

    for h in range(NSTAGE):
        pltpu.sync_copy(src_hbm.at[w, pl.ds(h * HSTAGE, HSTAGE)], src_v)
        pltpu.sync_copy(dst_hbm.at[w, pl.ds(h * HSTAGE, HSTAGE)], dst_v)
        for b in range(NBUF - 1):
            _gstart(b, b)

        def _step(j, carry):
            b = lax.rem(j, NBUF)
            _gwait(j, b)
            _sstart(j, b)

            @pl.when(j >= 1)
            def _():
                _swait(j - 1, lax.rem(j - 1, NBUF))

            @pl.when(j + NBUF - 1 < HSTAGE)
            def _():
                jn = j + NBUF - 1
                _gstart(jn, lax.rem(jn, NBUF))

            return carry

        lax.fori_loop(0, HSTAGE, _step, 0)
        _swait(HSTAGE - 1, (HSTAGE - 1) % NBUF)

    plsc.subcore_barrier()

    def _out(i, carry):
        r = row0 + i * CHUNK
        pltpu.sync_copy(acc.at[pl.ds(r, CHUNK)], out_hbm.at[c].at[pl.ds(r, CHUNK)])
        return carry

    lax.fori_loop(0, ROWS_PER_TILE // CHUNK, _out, 0)


def _tc_hp1(deg_ref, x_ref, w1_ref, hp_ref, dis_ref):
    deg = jnp.sum(deg_ref[...], axis=0) + 1.0
    dis = lax.rsqrt(deg)
    dis_ref[...] = dis
    h = jnp.dot(x_ref[...], w1_ref[...], preferred_element_type=jnp.float32)
    hp_ref[...] = h * dis[:, None]


def _tc_hp2(agg_ref, hp1_ref, dis_ref, b1_ref, w2_ref, hp2_ref):
    dis = dis_ref[...]
    h1 = (agg_ref[0] + agg_ref[1] + hp1_ref[...]) * dis[:, None] + b1_ref[...][None, :]
    h = jnp.dot(h1, w2_ref[...], preferred_element_type=jnp.float32)
    hp2_ref[...] = h * dis[:, None]


def _tc_out(agg_ref, hp2_ref, dis_ref, b2_ref, out_ref):
    dis = dis_ref[...]
    h = (agg_ref[0] + agg_ref[1] + hp2_ref[...]) * dis[:, None] + b2_ref[...][None, :]
    out_ref[...] = jnp.maximum(h, 0.0)


def kernel(x, edge_index, W1, b1, W2, b2):
    src = edge_index[0].astype(jnp.int32)
    dst = edge_index[1].astype(jnp.int32)
    n_edges = src.shape[0]
    ppw = (EPAD - n_edges) // NW
    src_w = jnp.concatenate(
        [src.reshape(NW, n_edges // NW), jnp.zeros((NW, ppw), jnp.int32)], axis=1
    ).reshape(NW, CPW, CHUNK)
    dst_w = jnp.concatenate(
        [dst.reshape(NW, n_edges // NW), jnp.full((NW, ppw), N_NODES, jnp.int32)],
        axis=1,
    ).reshape(NW, CPW, CHUNK)

    deg_parts = _deg_kernel(dst_w)
    x_pad = jnp.pad(x, ((0, NPAD - N_NODES), (0, 0)))

    hp1, dis = pl.pallas_call(
        _tc_hp1,
        out_shape=[
            jax.ShapeDtypeStruct((NPAD, D), jnp.float32),
            jax.ShapeDtypeStruct((NPAD,), jnp.float32),
        ],
    )(deg_parts, x_pad, W1)

    zeros_acc = jnp.zeros((NPAD, D), jnp.float32)
    agg1 = _agg_kernel(hp1, src_w, dst_w, zeros_acc)

    hp2 = pl.pallas_call(
        _tc_hp2, out_shape=jax.ShapeDtypeStruct((NPAD, D), jnp.float32)
    )(agg1, hp1, dis, b1, W2)

    agg2 = _agg_kernel(hp2, src_w, dst_w, zeros_acc)

    out_full = pl.pallas_call(
        _tc_out, out_shape=jax.ShapeDtypeStruct((NPAD, D), jnp.float32)
    )(agg2, hp2, dis, b2)

    return out_full[:N_NODES]


# speedup vs baseline: 1.9224x; 1.9224x over previous
import functools

import jax
import jax.numpy as jnp
from jax import lax
from jax.experimental import pallas as pl
from jax.experimental.pallas import tpu as pltpu
from jax.experimental.pallas import tpu_sc as plsc

N_NODES = 10000
D = 128
NPAD = 10112
H = 5056
HALF_PAD = 5120
NC = 2
NS = 16
NW = NC * NS
CHUNK = 64
CPW = 160
NBUF = 2
EPW = CPW * CHUNK
EPAD = NW * EPW
QCH = 176
QCAP = QCH * CHUNK
GPW = EPW // 16
HP_TILE = 320
ACC_TILE = HALF_PAD // NS

_MESH = plsc.VectorSubcoreMesh(core_axis_name="c", subcore_axis_name="s")


@functools.partial(
    pl.kernel,
    out_type=(
        jax.ShapeDtypeStruct((NW, 4 * QCAP), jnp.int32),
        jax.ShapeDtypeStruct((NW, 4 * QCAP), jnp.int32),
        jax.ShapeDtypeStruct((NW, 4, 16), jnp.int32),
        jax.ShapeDtypeStruct((NW, NPAD), jnp.float32),
    ),
    mesh=_MESH,
    scratch_types=[
        pltpu.VMEM((CPW, CHUNK), jnp.int32),
        pltpu.VMEM((CPW, CHUNK), jnp.int32),
        pltpu.VMEM((2 * QCAP,), jnp.int32),
        pltpu.VMEM((2 * QCAP,), jnp.int32),
        pltpu.VMEM((NPAD,), jnp.float32),
        pltpu.VMEM((4, 16), jnp.int32),
    ],
    compiler_params=pltpu.CompilerParams(needs_layout_passes=False),
)
def _part_kernel(src_hbm, dst_hbm, qsrc_out, qdst_out, nch_out, deg_out,
                 src_v, dst_v, qs_v, qd_v, hist_v, cnt_v):
    c = lax.axis_index("c")
    s = lax.axis_index("s")
    w = s * NC + c
    pltpu.sync_copy(src_hbm.at[w], src_v)
    pltpu.sync_copy(dst_hbm.at[w], dst_v)

    def _zero(i, carry):
        hist_v[pl.ds(i * 16, 16)] = jnp.zeros((16,), jnp.float32)
        return carry

    lax.fori_loop(0, NPAD // 16, _zero, 0)

    ones = jnp.ones((16,), jnp.float32)
    hconst = jnp.full((16,), H, jnp.int32)
    dumpd = jnp.full((16,), H, jnp.int32) + w
    zeros16 = jnp.zeros((16,), jnp.int32)

    for p in range(2):
        def _group(g, cnts):
            j = g // (CHUNK // 16)
            k = g % (CHUNK // 16)
            s16 = src_v[j, pl.ds(k * 16, 16)]
            d16 = dst_v[j, pl.ds(k * 16, 16)]
            if p == 0:
                plsc.addupdate_scatter(hist_v, [d16], ones)
            sa = s16 < hconst
            da = d16 < hconst
            sm = sa if p == 0 else jnp.logical_not(sa)
            new = []
            for qi in range(2):
                m = jnp.logical_and(
                    sm, da if qi == 0 else jnp.logical_not(da))
                ls = s16 - p * H
                ld = d16 - qi * H
                off = qi * QCAP + cnts[qi]
                plsc.store_compressed(qs_v.at[pl.ds(off, 16)], ls, mask=m)
                plsc.store_compressed(qd_v.at[pl.ds(off, 16)], ld, mask=m)
                inc = jnp.max(plsc.all_reduce_population_count(m))
                new.append(cnts[qi] + inc)
            return tuple(new)

        z = jnp.int32(0)
        cnts = lax.fori_loop(0, GPW, _group, (z, z))

        for qi in range(2):
            off = qi * QCAP + cnts[qi]
            for t in range(CHUNK // 16):
                qs_v[pl.ds(off + t * 16, 16)] = zeros16
                qd_v[pl.ds(off + t * 16, 16)] = dumpd
            nch = (cnts[qi] + (CHUNK - 1)) // CHUNK
            cnt_v[2 * p + qi, pl.ds(0, 16)] = jnp.full((16,), 1, jnp.int32) * nch

        pltpu.sync_copy(qs_v, qsrc_out.at[w, pl.ds(p * 2 * QCAP, 2 * QCAP)])
        pltpu.sync_copy(qd_v, qdst_out.at[w, pl.ds(p * 2 * QCAP, 2 * QCAP)])

    pltpu.sync_copy(cnt_v, nch_out.at[w])
    pltpu.sync_copy(hist_v, deg_out.at[w])


def _make_agg(dhalf):
    @functools.partial(
        pl.kernel,
        out_type=jax.ShapeDtypeStruct((NC, HALF_PAD, D), jnp.float32),
        mesh=_MESH,
        scratch_types=[
            pltpu.VMEM((QCH // 2, CHUNK), jnp.int32),
            pltpu.VMEM((QCH // 2, CHUNK), jnp.int32),
            pltpu.VMEM((16,), jnp.int32),
            pltpu.VMEM((NBUF, CHUNK, D), jnp.float32),
            pltpu.VMEM_SHARED((H, D), jnp.float32),
            pltpu.VMEM_SHARED((HALF_PAD, D), jnp.float32),
            pltpu.SemaphoreType.DMA,
            pltpu.SemaphoreType.DMA,
        ],
        compiler_params=pltpu.CompilerParams(needs_layout_passes=False),
    )
    def _agg(hp_hbm, qsrc_hbm, qdst_hbm, nch_hbm, zeros_hbm, out_hbm,
             sidx_v, didx_v, cnt_v, bufs, hp_s, acc, gsem, ssem):
        c = lax.axis_index("c")
        s = lax.axis_index("s")

        @pl.when(s < NS - 1)
        def _():
            pltpu.sync_copy(
                hp_hbm.at[pl.ds(c * H + s * HP_TILE, HP_TILE)],
                hp_s.at[pl.ds(s * HP_TILE, HP_TILE)],
            )

        @pl.when(s == NS - 1)
        def _():
            pltpu.sync_copy(
                hp_hbm.at[pl.ds(c * H + (NS - 1) * HP_TILE, H - (NS - 1) * HP_TILE)],
                hp_s.at[pl.ds((NS - 1) * HP_TILE, H - (NS - 1) * HP_TILE)],
            )
        pltpu.sync_copy(
            zeros_hbm.at[pl.ds(s * ACC_TILE, ACC_TILE)],
            acc.at[pl.ds(s * ACC_TILE, ACC_TILE)],
        )
        plsc.subcore_barrier()

        def _gstart(j, b):
            pltpu.async_copy(hp_s.at[sidx_v.at[j]], bufs.at[b], gsem)

        def _gwait(j, b):
            pltpu.make_async_copy(hp_s.at[sidx_v.at[j]], bufs.at[b], gsem).wait()

        def _sstart(j, b):
            pltpu.async_copy(bufs.at[b], acc.at[didx_v.at[j]], ssem, add=True)

        def _swait(j, b):
            pltpu.make_async_copy(bufs.at[b], acc.at[didx_v.at[j]], ssem).wait()

        q = 2 * c + dhalf
        hstg = QCH // 2
        for w2i in range(2):
            w2 = 2 * s + w2i
            pltpu.sync_copy(nch_hbm.at[w2, q], cnt_v)
            nch = jnp.max(cnt_v[pl.ds(0, 16)])
            for part in range(2):
                base = part * hstg
                npart = jnp.minimum(jnp.maximum(nch - base, 0), hstg)

                @pl.when(npart > 0)
                def _():
                    pltpu.sync_copy(
                        qsrc_hbm.at[w2, q, pl.ds(base, hstg)], sidx_v)
                    pltpu.sync_copy(
                        qdst_hbm.at[w2, q, pl.ds(base, hstg)], didx_v)

                    for b in range(NBUF - 1):
                        @pl.when(b < npart)
                        def _():
                            _gstart(b, b)

                    def _step(j, carry):
                        b = lax.rem(j, NBUF)
                        _gwait(j, b)
                        _sstart(j, b)

                        @pl.when(j >= 1)
                        def _():
                            _swait(j - 1, lax.rem(j - 1, NBUF))

                        @pl.when(j + NBUF - 1 < npart)
                        def _():
                            jn = j + NBUF - 1
                            _gstart(jn, lax.rem(jn, NBUF))

                        return carry

                    lax.fori_loop(0, npart, _step, 0)
                    _swait(npart - 1, lax.rem(npart - 1, NBUF))

        plsc.subcore_barrier()
        pltpu.sync_copy(
            acc.at[pl.ds(s * ACC_TILE, ACC_TILE)],
            out_hbm.at[c].at[pl.ds(s * ACC_TILE, ACC_TILE)],
        )

    return _agg


_agg_half = (_make_agg(0), _make_agg(1))


def _tc_hp1(deg_ref, x_ref, w1_ref, hp_ref, dis_ref):
    deg = jnp.sum(deg_ref[...], axis=0) + 1.0
    dis = lax.rsqrt(deg)
    dis_ref[...] = dis
    h = jnp.dot(x_ref[...], w1_ref[...], preferred_element_type=jnp.float32)
    hp_ref[...] = h * dis[:, None]


def _combine(agg_a, agg_b):
    return jnp.concatenate(
        [agg_a[0, :H] + agg_a[1, :H], agg_b[0, :H] + agg_b[1, :H]], axis=0
    )


def _tc_hp2(agga_ref, aggb_ref, hp1_ref, dis_ref, b1_ref, w2_ref, hp2_ref):
    dis = dis_ref[...]
    agg = _combine(agga_ref[...], aggb_ref[...])
    h1 = (agg + hp1_ref[...]) * dis[:, None] + b1_ref[...][None, :]
    h = jnp.dot(h1, w2_ref[...], preferred_element_type=jnp.float32)
    hp2_ref[...] = h * dis[:, None]


def _tc_out(agga_ref, aggb_ref, hp2_ref, dis_ref, b2_ref, out_ref):
    dis = dis_ref[...]
    agg = _combine(agga_ref[...], aggb_ref[...])
    h = (agg + hp2_ref[...]) * dis[:, None] + b2_ref[...][None, :]
    out_ref[...] = jnp.maximum(h, 0.0)


def kernel(x, edge_index, W1, b1, W2, b2):
    src = edge_index[0].astype(jnp.int32)
    dst = edge_index[1].astype(jnp.int32)
    n_edges = src.shape[0]
    ppw = (EPAD - n_edges) // NW
    src_w = jnp.concatenate(
        [src.reshape(NW, n_edges // NW), jnp.zeros((NW, ppw), jnp.int32)], axis=1
    ).reshape(NW, CPW, CHUNK)
    dst_w = jnp.concatenate(
        [dst.reshape(NW, n_edges // NW), jnp.full((NW, ppw), N_NODES, jnp.int32)],
        axis=1,
    ).reshape(NW, CPW, CHUNK)

    qsrc, qdst, nch, deg_parts = _part_kernel(src_w, dst_w)
    qsrc = qsrc.reshape(NW, 4, QCH, CHUNK)
    qdst = qdst.reshape(NW, 4, QCH, CHUNK)

    x_pad = jnp.pad(x, ((0, NPAD - N_NODES), (0, 0)))
    hp1, dis = pl.pallas_call(
        _tc_hp1,
        out_shape=[
            jax.ShapeDtypeStruct((NPAD, D), jnp.float32),
            jax.ShapeDtypeStruct((NPAD,), jnp.float32),
        ],
    )(deg_parts, x_pad, W1)

    zeros_acc = jnp.zeros((HALF_PAD, D), jnp.float32)
    agg1a = _agg_half[0](hp1, qsrc, qdst, nch, zeros_acc)
    agg1b = _agg_half[1](hp1, qsrc, qdst, nch, zeros_acc)

    hp2 = pl.pallas_call(
        _tc_hp2, out_shape=jax.ShapeDtypeStruct((NPAD, D), jnp.float32)
    )(agg1a, agg1b, hp1, dis, b1, W2)

    agg2a = _agg_half[0](hp2, qsrc, qdst, nch, zeros_acc)
    agg2b = _agg_half[1](hp2, qsrc, qdst, nch, zeros_acc)

    out_full = pl.pallas_call(
        _tc_out, out_shape=jax.ShapeDtypeStruct((NPAD, D), jnp.float32)
    )(agg2a, agg2b, hp2, dis, b2)

    return out_full[:N_NODES]


# speedup vs baseline: 1.9972x; 1.0389x over previous
import functools

import jax
import jax.numpy as jnp
from jax import lax
from jax.experimental import pallas as pl
from jax.experimental.pallas import tpu as pltpu
from jax.experimental.pallas import tpu_sc as plsc

N_NODES = 10000
D = 128
NPAD = 10112
H = 5056
HALF_PAD = 5120
NC = 2
NS = 16
NW = NC * NS
CHUNK = 64
CPW = 160
NBUF = 2
EPW = CPW * CHUNK
EPAD = NW * EPW
QCH = 176
QCAP = QCH * CHUNK
GPW = EPW // 16
HP_TILE = 320
ACC_TILE = HALF_PAD // NS

_MESH = plsc.VectorSubcoreMesh(core_axis_name="c", subcore_axis_name="s")


@functools.partial(
    pl.kernel,
    out_type=(
        jax.ShapeDtypeStruct((NW, 4 * QCAP), jnp.int32),
        jax.ShapeDtypeStruct((NW, 4 * QCAP), jnp.int32),
        jax.ShapeDtypeStruct((NW, 4, 16), jnp.int32),
        jax.ShapeDtypeStruct((NW, NPAD), jnp.float32),
    ),
    mesh=_MESH,
    scratch_types=[
        pltpu.VMEM((CPW, CHUNK), jnp.int32),
        pltpu.VMEM((CPW, CHUNK), jnp.int32),
        pltpu.VMEM((2 * QCAP,), jnp.int32),
        pltpu.VMEM((2 * QCAP,), jnp.int32),
        pltpu.VMEM((NPAD,), jnp.float32),
        pltpu.VMEM((4, 16), jnp.int32),
    ],
    compiler_params=pltpu.CompilerParams(needs_layout_passes=False),
)
def _part_kernel(src_hbm, dst_hbm, qsrc_out, qdst_out, nch_out, deg_out,
                 src_v, dst_v, qs_v, qd_v, hist_v, cnt_v):
    c = lax.axis_index("c")
    s = lax.axis_index("s")
    w = s * NC + c
    pltpu.sync_copy(src_hbm.at[w], src_v)
    pltpu.sync_copy(dst_hbm.at[w], dst_v)

    def _zero(i, carry):
        hist_v[pl.ds(i * 16, 16)] = jnp.zeros((16,), jnp.float32)
        return carry

    lax.fori_loop(0, NPAD // 16, _zero, 0)

    ones = jnp.ones((16,), jnp.float32)
    hconst = jnp.full((16,), H, jnp.int32)
    dumpd = jnp.full((16,), H, jnp.int32) + w
    zeros16 = jnp.zeros((16,), jnp.int32)

    for p in range(2):
        def _group(g, cnts):
            j = g // (CHUNK // 16)
            k = g % (CHUNK // 16)
            s16 = src_v[j, pl.ds(k * 16, 16)]
            d16 = dst_v[j, pl.ds(k * 16, 16)]
            if p == 0:
                plsc.addupdate_scatter(hist_v, [d16], ones)
            sa = s16 < hconst
            da = d16 < hconst
            sm = sa if p == 0 else jnp.logical_not(sa)
            new = []
            for qi in range(2):
                m = jnp.logical_and(
                    sm, da if qi == 0 else jnp.logical_not(da))
                ls = s16 - p * H
                ld = d16 - qi * H
                off = qi * QCAP + cnts[qi]
                plsc.store_compressed(qs_v.at[pl.ds(off, 16)], ls, mask=m)
                plsc.store_compressed(qd_v.at[pl.ds(off, 16)], ld, mask=m)
                inc = jnp.max(plsc.all_reduce_population_count(m))
                new.append(cnts[qi] + inc)
            return tuple(new)

        z = jnp.int32(0)
        cnts = lax.fori_loop(0, GPW, _group, (z, z))

        for qi in range(2):
            off = qi * QCAP + cnts[qi]
            for t in range(CHUNK // 16):
                qs_v[pl.ds(off + t * 16, 16)] = zeros16
                qd_v[pl.ds(off + t * 16, 16)] = dumpd
            nch = (cnts[qi] + (CHUNK - 1)) // CHUNK
            cnt_v[2 * p + qi, pl.ds(0, 16)] = jnp.full((16,), 1, jnp.int32) * nch

        pltpu.sync_copy(qs_v, qsrc_out.at[w, pl.ds(p * 2 * QCAP, 2 * QCAP)])
        pltpu.sync_copy(qd_v, qdst_out.at[w, pl.ds(p * 2 * QCAP, 2 * QCAP)])

    pltpu.sync_copy(cnt_v, nch_out.at[w])
    pltpu.sync_copy(hist_v, deg_out.at[w])


def _make_agg():
    @functools.partial(
        pl.kernel,
        out_type=jax.ShapeDtypeStruct((NC, 2, HALF_PAD, D), jnp.float32),
        mesh=_MESH,
        scratch_types=[
            pltpu.VMEM((QCH // 2, CHUNK), jnp.int32),
            pltpu.VMEM((QCH // 2, CHUNK), jnp.int32),
            pltpu.VMEM((16,), jnp.int32),
            pltpu.VMEM((NBUF, CHUNK, D), jnp.float32),
            pltpu.VMEM_SHARED((H, D), jnp.float32),
            pltpu.VMEM_SHARED((HALF_PAD, D), jnp.float32),
            pltpu.SemaphoreType.DMA,
            pltpu.SemaphoreType.DMA,
        ],
        compiler_params=pltpu.CompilerParams(needs_layout_passes=False),
    )
    def _agg(hp_hbm, qsrc_hbm, qdst_hbm, nch_hbm, zeros_hbm, out_hbm,
             sidx_v, didx_v, cnt_v, bufs, hp_s, acc, gsem, ssem):
        c = lax.axis_index("c")
        s = lax.axis_index("s")

        @pl.when(s < NS - 1)
        def _():
            pltpu.sync_copy(
                hp_hbm.at[pl.ds(c * H + s * HP_TILE, HP_TILE)],
                hp_s.at[pl.ds(s * HP_TILE, HP_TILE)],
            )

        @pl.when(s == NS - 1)
        def _():
            pltpu.sync_copy(
                hp_hbm.at[pl.ds(c * H + (NS - 1) * HP_TILE, H - (NS - 1) * HP_TILE)],
                hp_s.at[pl.ds((NS - 1) * HP_TILE, H - (NS - 1) * HP_TILE)],
            )

        def _gstart(j, b):
            pltpu.async_copy(hp_s.at[sidx_v.at[j]], bufs.at[b], gsem)

        def _gwait(j, b):
            pltpu.make_async_copy(hp_s.at[sidx_v.at[j]], bufs.at[b], gsem).wait()

        def _sstart(j, b):
            pltpu.async_copy(bufs.at[b], acc.at[didx_v.at[j]], ssem, add=True)

        def _swait(j, b):
            pltpu.make_async_copy(bufs.at[b], acc.at[didx_v.at[j]], ssem).wait()

        hstg = QCH // 2

        def _run_queues(q):
            for w2i in range(2):
                w2 = 2 * s + w2i
                pltpu.sync_copy(nch_hbm.at[w2, q], cnt_v)
                nch = jnp.max(cnt_v[pl.ds(0, 16)])
                for part in range(2):
                    base = part * hstg
                    npart = jnp.minimum(jnp.maximum(nch - base, 0), hstg)

                    @pl.when(npart > 0)
                    def _():
                        pltpu.sync_copy(
                            qsrc_hbm.at[w2, q, pl.ds(base, hstg)], sidx_v)
                        pltpu.sync_copy(
                            qdst_hbm.at[w2, q, pl.ds(base, hstg)], didx_v)

                        for b in range(NBUF - 1):
                            @pl.when(b < npart)
                            def _():
                                _gstart(b, b)

                        def _step(j, carry):
                            b = lax.rem(j, NBUF)
                            _gwait(j, b)
                            _sstart(j, b)

                            @pl.when(j >= 1)
                            def _():
                                _swait(j - 1, lax.rem(j - 1, NBUF))

                            @pl.when(j + NBUF - 1 < npart)
                            def _():
                                jn = j + NBUF - 1
                                _gstart(jn, lax.rem(jn, NBUF))

                            return carry

                        lax.fori_loop(0, npart, _step, 0)
                        _swait(npart - 1, lax.rem(npart - 1, NBUF))

        for dhalf in range(2):
            q = 2 * c + dhalf
            pltpu.sync_copy(
                zeros_hbm.at[pl.ds(s * ACC_TILE, ACC_TILE)],
                acc.at[pl.ds(s * ACC_TILE, ACC_TILE)],
            )
            plsc.subcore_barrier()
            _run_queues(q)
            plsc.subcore_barrier()
            pltpu.sync_copy(
                acc.at[pl.ds(s * ACC_TILE, ACC_TILE)],
                out_hbm.at[c, dhalf].at[pl.ds(s * ACC_TILE, ACC_TILE)],
            )

    return _agg


_agg_kernel = _make_agg()


def _tc_hp1(deg_ref, x_ref, w1_ref, hp_ref, dis_ref):
    deg = jnp.sum(deg_ref[...], axis=0) + 1.0
    dis = lax.rsqrt(deg)
    dis_ref[...] = dis
    h = jnp.dot(x_ref[...], w1_ref[...], preferred_element_type=jnp.float32)
    hp_ref[...] = h * dis[:, None]


def _combine(agg):
    return jnp.concatenate(
        [agg[0, 0, :H] + agg[1, 0, :H], agg[0, 1, :H] + agg[1, 1, :H]], axis=0
    )


def _tc_hp2(agg_ref, hp1_ref, dis_ref, b1_ref, w2_ref, hp2_ref):
    dis = dis_ref[...]
    agg = _combine(agg_ref[...])
    h1 = (agg + hp1_ref[...]) * dis[:, None] + b1_ref[...][None, :]
    h = jnp.dot(h1, w2_ref[...], preferred_element_type=jnp.float32)
    hp2_ref[...] = h * dis[:, None]


def _tc_out(agg_ref, hp2_ref, dis_ref, b2_ref, out_ref):
    dis = dis_ref[...]
    agg = _combine(agg_ref[...])
    h = (agg + hp2_ref[...]) * dis[:, None] + b2_ref[...][None, :]
    out_ref[...] = jnp.maximum(h, 0.0)


def kernel(x, edge_index, W1, b1, W2, b2):
    src = edge_index[0].astype(jnp.int32)
    dst = edge_index[1].astype(jnp.int32)
    n_edges = src.shape[0]
    ppw = (EPAD - n_edges) // NW
    src_w = jnp.concatenate(
        [src.reshape(NW, n_edges // NW), jnp.zeros((NW, ppw), jnp.int32)], axis=1
    ).reshape(NW, CPW, CHUNK)
    dst_w = jnp.concatenate(
        [dst.reshape(NW, n_edges // NW), jnp.full((NW, ppw), N_NODES, jnp.int32)],
        axis=1,
    ).reshape(NW, CPW, CHUNK)

    qsrc, qdst, nch, deg_parts = _part_kernel(src_w, dst_w)
    qsrc = qsrc.reshape(NW, 4, QCH, CHUNK)
    qdst = qdst.reshape(NW, 4, QCH, CHUNK)

    x_pad = jnp.pad(x, ((0, NPAD - N_NODES), (0, 0)))
    hp1, dis = pl.pallas_call(
        _tc_hp1,
        out_shape=[
            jax.ShapeDtypeStruct((NPAD, D), jnp.float32),
            jax.ShapeDtypeStruct((NPAD,), jnp.float32),
        ],
    )(deg_parts, x_pad, W1)

    zeros_acc = jnp.zeros((HALF_PAD, D), jnp.float32)
    agg1 = _agg_kernel(hp1, qsrc, qdst, nch, zeros_acc)

    hp2 = pl.pallas_call(
        _tc_hp2, out_shape=jax.ShapeDtypeStruct((NPAD, D), jnp.float32)
    )(agg1, hp1, dis, b1, W2)

    agg2 = _agg_kernel(hp2, qsrc, qdst, nch, zeros_acc)

    out_full = pl.pallas_call(
        _tc_out, out_shape=jax.ShapeDtypeStruct((NPAD, D), jnp.float32)
    )(agg2, hp2, dis, b2)

    return out_full[:N_NODES]
